# Initial kernel scaffold; baseline (speedup 1.0000x reference)
import jax
import jax.numpy as jnp
from jax.experimental import pallas as pl


def kernel(node_feat, edge_feat, edge_weight, edge_index, g_W0, g_b0, g_W1, g_b1, g_W2, g_b2, o_W0, o_b0, o_W1, o_b1, o_W2, o_b2, lin_W, lin_b):
    raise NotImplementedError("write your pallas kernel here")


# speedup vs baseline: 2.1513x; 2.1513x over previous
import functools

import jax
import jax.numpy as jnp
from jax import lax
from jax.experimental import pallas as pl
from jax.experimental.pallas import tpu as pltpu
from jax.experimental.pallas import tpu_sc as plsc

N_NODES = 10000
N_EDGES = 320000
D = 128
ED = 16
H = 256

NC = 2
NS = 16
NW = NC * NS

GCH = 128


def _sc_gather_body(table_hbm, idx_hbm, out_hbm, idx_v, rows_v, sem):
    c = lax.axis_index("c")
    s = lax.axis_index("s")
    wid = s * NC + c
    n_total = idx_hbm.shape[0]
    per_w = n_total // NW
    base = wid * per_w

    @pl.loop(0, per_w // GCH)
    def _(j):
        off = pl.multiple_of(base + j * GCH, GCH)
        pltpu.sync_copy(idx_hbm.at[pl.ds(off, GCH)], idx_v)
        pltpu.async_copy(table_hbm.at[idx_v], rows_v, sem).wait()
        pltpu.sync_copy(rows_v, out_hbm.at[pl.ds(off, GCH)])


def _sc_gather(table, idx):
    b = idx.shape[0]
    mesh = plsc.VectorSubcoreMesh(core_axis_name="c", subcore_axis_name="s")
    return pl.kernel(
        _sc_gather_body,
        out_type=jax.ShapeDtypeStruct((b, table.shape[1]), table.dtype),
        mesh=mesh,
        scratch_types=[
            pltpu.VMEM((GCH,), jnp.int32),
            pltpu.VMEM((GCH, table.shape[1]), table.dtype),
            pltpu.SemaphoreType.DMA,
        ],
    )(table, idx)


SCH = 80
N_PAD = 10240
STRIPE = N_PAD // NS


def _sc_scatter_body(msg_hbm, dst_hbm, zeros_hbm, out_hbm, idx_v, rows_v, acc_sh):
    c = lax.axis_index("c")
    s = lax.axis_index("s")
    per_core = N_EDGES // NC
    per_tile = per_core // NS
    base = c * per_core + s * per_tile
    n_sub = STRIPE // SCH

    pltpu.sync_copy(zeros_hbm, rows_v)

    @pl.loop(0, n_sub)
    def _(k):
        off = pl.multiple_of(s * STRIPE + k * SCH, 8)
        pltpu.sync_copy(rows_v, acc_sh.at[pl.ds(off, SCH)])

    plsc.subcore_barrier()

    @pl.loop(0, per_tile // SCH)
    def _(j):
        off = pl.multiple_of(base + j * SCH, 8)
        pltpu.sync_copy(dst_hbm.at[pl.ds(off, SCH)], idx_v)
        pltpu.sync_copy(msg_hbm.at[pl.ds(off, SCH)], rows_v)
        pltpu.sync_copy(rows_v, acc_sh.at[idx_v], add=True)

    plsc.subcore_barrier()

    @pl.loop(0, n_sub)
    def _(k):
        off = pl.multiple_of(s * STRIPE + k * SCH, 8)
        pltpu.sync_copy(acc_sh.at[pl.ds(off, SCH)], rows_v)
        pltpu.sync_copy(rows_v, out_hbm.at[c, pl.ds(off, SCH)])


def _sc_segment_sum(msg, dst_idx, zeros_stripe):
    mesh = plsc.VectorSubcoreMesh(core_axis_name="c", subcore_axis_name="s")
    return pl.kernel(
        _sc_scatter_body,
        out_type=jax.ShapeDtypeStruct((NC, N_PAD, D), jnp.float32),
        mesh=mesh,
        scratch_types=[
            pltpu.VMEM((SCH,), jnp.int32),
            pltpu.VMEM((SCH, D), jnp.float32),
            pltpu.VMEM_SHARED((N_PAD, D), jnp.float32),
        ],
    )(msg, dst_idx, zeros_stripe)


BLK = 512


def _silu(x):
    return x * jax.nn.sigmoid(x)


def _mlp_body(src, dst, ef, ew,
              w0s, w0d, w0e, b0, gw1, gb1, ow1, ob1, gw2, gb2, ow2, ob2,
              msg_out):
    f32 = jnp.float32
    pre0 = (jnp.dot(src[...], w0s[...], preferred_element_type=f32)
            + jnp.dot(dst[...], w0d[...], preferred_element_type=f32)
            + jnp.dot(ef[...], w0e[...], preferred_element_type=f32)
            + b0[...])
    a1 = _silu(pre0)
    g1 = _silu(jnp.dot(a1[:, :H], gw1[...], preferred_element_type=f32) + gb1[...])
    o1 = _silu(jnp.dot(a1[:, H:], ow1[...], preferred_element_type=f32) + ob1[...])
    gp = jnp.dot(g1, gw2[...], preferred_element_type=f32) + gb2[...]
    op = jnp.dot(o1, ow2[...], preferred_element_type=f32) + ob2[...]
    msg_out[...] = _silu(op) * jax.nn.sigmoid(gp) * ew[...]


def _edge_mlp(gathered, edge_feat, edge_weight, weights):
    n_blocks = N_EDGES // BLK
    dst_block_off = N_EDGES // BLK

    def full(w):
        return pl.BlockSpec(w.shape, lambda i: tuple(0 for _ in w.shape))

    w_specs = [full(w) for w in weights]
    return pl.pallas_call(
        _mlp_body,
        grid=(n_blocks,),
        in_specs=[
            pl.BlockSpec((BLK, D), lambda i: (i, 0)),
            pl.BlockSpec((BLK, D), lambda i: (i + dst_block_off, 0)),
            pl.BlockSpec((BLK, ED), lambda i: (i, 0)),
            pl.BlockSpec((BLK, D), lambda i: (i, 0)),
            *w_specs,
        ],
        out_specs=pl.BlockSpec((BLK, D), lambda i: (i, 0)),
        out_shape=jax.ShapeDtypeStruct((N_EDGES, D), jnp.float32),
    )(gathered, gathered, edge_feat, edge_weight, *weights)


NBLK = 2000


def _final_body(node_feat, partials, lin_w, lin_b, out):
    agg = partials[0] + partials[1]
    out[...] = node_feat[...] + jnp.dot(
        agg, lin_w[...], preferred_element_type=jnp.float32) + lin_b[...]


def _final_linear(node_feat, partials, lin_w, lin_b):
    n_blocks = N_NODES // NBLK
    return pl.pallas_call(
        _final_body,
        grid=(n_blocks,),
        in_specs=[
            pl.BlockSpec((NBLK, D), lambda i: (i, 0)),
            pl.BlockSpec((NC, NBLK, D), lambda i: (0, i, 0)),
            pl.BlockSpec((D, D), lambda i: (0, 0)),
            pl.BlockSpec((1, D), lambda i: (0, 0)),
        ],
        out_specs=pl.BlockSpec((NBLK, D), lambda i: (i, 0)),
        out_shape=jax.ShapeDtypeStruct((N_NODES, D), jnp.float32),
    )(node_feat, partials, lin_w, lin_b)


def kernel(node_feat, edge_feat, edge_weight, edge_index,
           g_W0, g_b0, g_W1, g_b1, g_W2, g_b2,
           o_W0, o_b0, o_W1, o_b1, o_W2, o_b2,
           lin_W, lin_b):
    idx_flat = edge_index.reshape(-1).astype(jnp.int32)
    b_pad = NW * GCH * ((2 * N_EDGES + NW * GCH - 1) // (NW * GCH))
    idx_pad = jnp.concatenate(
        [idx_flat, jnp.zeros((b_pad - 2 * N_EDGES,), jnp.int32)])
    gathered = _sc_gather(node_feat, idx_pad)

    w0 = jnp.concatenate([g_W0, o_W0], axis=1)
    w0s, w0d, w0e = w0[:D], w0[D:2 * D], w0[2 * D:]
    b0 = jnp.concatenate([g_b0, o_b0]).reshape(1, 2 * H)
    weights = [w0s, w0d, w0e, b0,
               g_W1, g_b1.reshape(1, H), o_W1, o_b1.reshape(1, H),
               g_W2, g_b2.reshape(1, D), o_W2, o_b2.reshape(1, D)]

    msg = _edge_mlp(gathered, edge_feat, edge_weight, weights)

    dst_idx = edge_index[1].astype(jnp.int32)
    zeros_stripe = jnp.zeros((SCH, D), jnp.float32)
    partials = _sc_segment_sum(msg, dst_idx, zeros_stripe)

    return _final_linear(node_feat, partials, lin_W, lin_b.reshape(1, D))


# speedup vs baseline: 2.3263x; 1.0813x over previous
import functools

import jax
import jax.numpy as jnp
from jax import lax
from jax.experimental import pallas as pl
from jax.experimental.pallas import tpu as pltpu
from jax.experimental.pallas import tpu_sc as plsc

N_NODES = 10000
N_EDGES = 320000
D = 128
ED = 16
H = 256

NC = 2
NS = 16
NW = NC * NS

GCH = 128


def _sc_gather_body(table_hbm, idx_hbm, out_hbm,
                    idx_a, idx_b, rows_a, rows_b, sem_a, sem_b):
    c = lax.axis_index("c")
    s = lax.axis_index("s")
    wid = s * NC + c
    n_total = idx_hbm.shape[0]
    per_w = n_total // NW
    base = wid * per_w
    n_pairs = per_w // GCH // 2

    def chunk(j):
        return pl.ds(pl.multiple_of(base + j * GCH, GCH), GCH)

    pltpu.sync_copy(idx_hbm.at[chunk(0)], idx_a)
    pltpu.async_copy(table_hbm.at[idx_a], rows_a, sem_a)

    @pl.loop(0, n_pairs)
    def _(jj):
        j = jj * 2
        pltpu.sync_copy(idx_hbm.at[chunk(j + 1)], idx_b)
        pltpu.async_copy(table_hbm.at[idx_b], rows_b, sem_b)
        pltpu.make_async_copy(table_hbm.at[idx_a], rows_a, sem_a).wait()
        pltpu.sync_copy(rows_a, out_hbm.at[chunk(j)])

        @pl.when(jj < n_pairs - 1)
        def _():
            pltpu.sync_copy(idx_hbm.at[chunk(j + 2)], idx_a)
            pltpu.async_copy(table_hbm.at[idx_a], rows_a, sem_a)

        pltpu.make_async_copy(table_hbm.at[idx_b], rows_b, sem_b).wait()
        pltpu.sync_copy(rows_b, out_hbm.at[chunk(j + 1)])


def _sc_gather(table, idx):
    b = idx.shape[0]
    mesh = plsc.VectorSubcoreMesh(core_axis_name="c", subcore_axis_name="s")
    w = table.shape[1]
    return pl.kernel(
        _sc_gather_body,
        out_type=jax.ShapeDtypeStruct((b, w), table.dtype),
        mesh=mesh,
        compiler_params=pltpu.CompilerParams(use_tc_tiling_on_sc=False),
        scratch_types=[
            pltpu.VMEM((GCH,), jnp.int32),
            pltpu.VMEM((GCH,), jnp.int32),
            pltpu.VMEM((GCH, w), table.dtype),
            pltpu.VMEM((GCH, w), table.dtype),
            pltpu.SemaphoreType.DMA,
            pltpu.SemaphoreType.DMA,
        ],
    )(table, idx)


SCH = 80
N_PAD = 10240
STRIPE = N_PAD // NS


def _sc_scatter_body(msg_hbm, dst_hbm, zeros_hbm, out_hbm, idx_v, rows_v, acc_sh):
    c = lax.axis_index("c")
    s = lax.axis_index("s")
    per_core = N_EDGES // NC
    per_tile = per_core // NS
    base = c * per_core + s * per_tile
    n_sub = STRIPE // SCH

    pltpu.sync_copy(zeros_hbm, rows_v)

    @pl.loop(0, n_sub)
    def _(k):
        off = pl.multiple_of(s * STRIPE + k * SCH, 8)
        pltpu.sync_copy(rows_v, acc_sh.at[pl.ds(off, SCH)])

    plsc.subcore_barrier()

    @pl.loop(0, per_tile // SCH)
    def _(j):
        off = pl.multiple_of(base + j * SCH, 8)
        pltpu.sync_copy(dst_hbm.at[pl.ds(off, SCH)], idx_v)
        pltpu.sync_copy(msg_hbm.at[pl.ds(off, SCH)], rows_v)
        pltpu.sync_copy(rows_v, acc_sh.at[idx_v], add=True)

    plsc.subcore_barrier()

    @pl.loop(0, n_sub)
    def _(k):
        off = pl.multiple_of(s * STRIPE + k * SCH, 8)
        pltpu.sync_copy(acc_sh.at[pl.ds(off, SCH)], rows_v)
        pltpu.sync_copy(rows_v, out_hbm.at[c, pl.ds(off, SCH)])


def _sc_segment_sum(msg, dst_idx, zeros_stripe):
    mesh = plsc.VectorSubcoreMesh(core_axis_name="c", subcore_axis_name="s")
    return pl.kernel(
        _sc_scatter_body,
        out_type=jax.ShapeDtypeStruct((NC, N_PAD, D), jnp.float32),
        mesh=mesh,
        scratch_types=[
            pltpu.VMEM((SCH,), jnp.int32),
            pltpu.VMEM((SCH, D), jnp.float32),
            pltpu.VMEM_SHARED((N_PAD, D), jnp.float32),
        ],
    )(msg, dst_idx, zeros_stripe)


BLK = 512


def _silu(x):
    return x * jax.nn.sigmoid(x)


def _unpack_bf16_pair(x_i32):
    f32 = jnp.float32
    bf = jnp.bfloat16
    even = lax.bitcast_convert_type(jnp.left_shift(x_i32, 16), f32)
    odd = lax.bitcast_convert_type(
        jnp.bitwise_and(x_i32, jnp.int32(-65536)), f32)
    return even.astype(bf), odd.astype(bf)


def _mlp_body(src, dst, ef, ew,
              w0sd, w0e, b0, gw1, gb1, ow1, ob1, gw2, gb2, ow2, ob2,
              msg_out):
    f32 = jnp.float32
    bf = jnp.bfloat16
    es, osrc = _unpack_bf16_pair(src[...])
    ed, od = _unpack_bf16_pair(dst[...])
    x = jnp.concatenate([es, osrc, ed, od], axis=1)
    pre0 = (jnp.dot(x, w0sd[...], preferred_element_type=f32)
            + jnp.dot(ef[...], w0e[...], preferred_element_type=f32)
            + b0[...])
    a1 = _silu(pre0).astype(bf)
    g1 = _silu(jnp.dot(a1[:, :H], gw1[...], preferred_element_type=f32) + gb1[...]).astype(bf)
    o1 = _silu(jnp.dot(a1[:, H:], ow1[...], preferred_element_type=f32) + ob1[...]).astype(bf)
    gp = jnp.dot(g1, gw2[...], preferred_element_type=f32) + gb2[...]
    op = jnp.dot(o1, ow2[...], preferred_element_type=f32) + ob2[...]
    msg_out[...] = _silu(op) * jax.nn.sigmoid(gp) * ew[...]


def _edge_mlp(gathered, edge_feat, edge_weight, weights):
    n_blocks = N_EDGES // BLK
    dst_block_off = N_EDGES // BLK

    def full(w):
        return pl.BlockSpec(w.shape, lambda i: tuple(0 for _ in w.shape))

    w_specs = [full(w) for w in weights]
    return pl.pallas_call(
        _mlp_body,
        grid=(n_blocks,),
        in_specs=[
            pl.BlockSpec((BLK, D // 2), lambda i: (i, 0)),
            pl.BlockSpec((BLK, D // 2), lambda i: (i + dst_block_off, 0)),
            pl.BlockSpec((BLK, ED), lambda i: (i, 0)),
            pl.BlockSpec((BLK, D), lambda i: (i, 0)),
            *w_specs,
        ],
        out_specs=pl.BlockSpec((BLK, D), lambda i: (i, 0)),
        out_shape=jax.ShapeDtypeStruct((N_EDGES, D), jnp.float32),
    )(gathered, gathered, edge_feat, edge_weight, *weights)


NBLK = 2000


def _final_body(node_feat, partials, lin_w, lin_b, out):
    agg = partials[0] + partials[1]
    out[...] = node_feat[...] + jnp.dot(
        agg, lin_w[...], preferred_element_type=jnp.float32) + lin_b[...]


def _final_linear(node_feat, partials, lin_w, lin_b):
    n_blocks = N_NODES // NBLK
    return pl.pallas_call(
        _final_body,
        grid=(n_blocks,),
        in_specs=[
            pl.BlockSpec((NBLK, D), lambda i: (i, 0)),
            pl.BlockSpec((NC, NBLK, D), lambda i: (0, i, 0)),
            pl.BlockSpec((D, D), lambda i: (0, 0)),
            pl.BlockSpec((1, D), lambda i: (0, 0)),
        ],
        out_specs=pl.BlockSpec((NBLK, D), lambda i: (i, 0)),
        out_shape=jax.ShapeDtypeStruct((N_NODES, D), jnp.float32),
    )(node_feat, partials, lin_w, lin_b)


def kernel(node_feat, edge_feat, edge_weight, edge_index,
           g_W0, g_b0, g_W1, g_b1, g_W2, g_b2,
           o_W0, o_b0, o_W1, o_b1, o_W2, o_b2,
           lin_W, lin_b):
    idx_flat = edge_index.reshape(-1).astype(jnp.int32)
    b_pad = NW * GCH * ((2 * N_EDGES + NW * GCH - 1) // (NW * GCH))
    idx_pad = jnp.concatenate(
        [idx_flat, jnp.zeros((b_pad - 2 * N_EDGES,), jnp.int32)])
    bf = jnp.bfloat16
    node_packed = lax.bitcast_convert_type(
        node_feat.astype(bf).reshape(N_NODES, D // 2, 2), jnp.int32)
    gathered = _sc_gather(node_packed, idx_pad)

    w0 = jnp.concatenate([g_W0, o_W0], axis=1)
    w0s, w0d, w0e = w0[:D], w0[D:2 * D], w0[2 * D:]
    w0sd = jnp.concatenate(
        [w0s[0::2], w0s[1::2], w0d[0::2], w0d[1::2]], axis=0)
    b0 = jnp.concatenate([g_b0, o_b0]).reshape(1, 2 * H)
    weights = [w0sd.astype(bf), w0e.astype(bf), b0,
               g_W1.astype(bf), g_b1.reshape(1, H), o_W1.astype(bf), o_b1.reshape(1, H),
               g_W2.astype(bf), g_b2.reshape(1, D), o_W2.astype(bf), o_b2.reshape(1, D)]

    msg = _edge_mlp(gathered, edge_feat.astype(bf), edge_weight, weights)

    dst_idx = edge_index[1].astype(jnp.int32)
    zeros_stripe = jnp.zeros((SCH, D), jnp.float32)
    partials = _sc_segment_sum(msg, dst_idx, zeros_stripe)

    return _final_linear(node_feat, partials, lin_W, lin_b.reshape(1, D))


# speedup vs baseline: 2.4635x; 1.0590x over previous
import functools

import jax
import jax.numpy as jnp
from jax import lax
from jax.experimental import pallas as pl
from jax.experimental.pallas import tpu as pltpu
from jax.experimental.pallas import tpu_sc as plsc

N_NODES = 10000
N_EDGES = 320000
D = 128
ED = 16
H = 256

NC = 2
NS = 16
NW = NC * NS

GCH = 128


def _sc_gather_body(table_hbm, idx_hbm, out_hbm,
                    idx_a, idx_b, rows_a, rows_b, sem_a, sem_b):
    c = lax.axis_index("c")
    s = lax.axis_index("s")
    wid = s * NC + c
    n_total = idx_hbm.shape[0]
    per_w = n_total // NW
    base = wid * per_w
    n_pairs = per_w // GCH // 2

    def chunk(j):
        return pl.ds(pl.multiple_of(base + j * GCH, GCH), GCH)

    pltpu.sync_copy(idx_hbm.at[chunk(0)], idx_a)
    pltpu.async_copy(table_hbm.at[idx_a], rows_a, sem_a)

    @pl.loop(0, n_pairs)
    def _(jj):
        j = jj * 2
        pltpu.sync_copy(idx_hbm.at[chunk(j + 1)], idx_b)
        pltpu.async_copy(table_hbm.at[idx_b], rows_b, sem_b)
        pltpu.make_async_copy(table_hbm.at[idx_a], rows_a, sem_a).wait()
        pltpu.sync_copy(rows_a, out_hbm.at[chunk(j)])

        @pl.when(jj < n_pairs - 1)
        def _():
            pltpu.sync_copy(idx_hbm.at[chunk(j + 2)], idx_a)
            pltpu.async_copy(table_hbm.at[idx_a], rows_a, sem_a)

        pltpu.make_async_copy(table_hbm.at[idx_b], rows_b, sem_b).wait()
        pltpu.sync_copy(rows_b, out_hbm.at[chunk(j + 1)])


def _sc_gather(table, idx):
    b = idx.shape[0]
    mesh = plsc.VectorSubcoreMesh(core_axis_name="c", subcore_axis_name="s")
    w = table.shape[1]
    return pl.kernel(
        _sc_gather_body,
        out_type=jax.ShapeDtypeStruct((b, w), table.dtype),
        mesh=mesh,
        compiler_params=pltpu.CompilerParams(use_tc_tiling_on_sc=False),
        scratch_types=[
            pltpu.VMEM((GCH,), jnp.int32),
            pltpu.VMEM((GCH,), jnp.int32),
            pltpu.VMEM((GCH, w), table.dtype),
            pltpu.VMEM((GCH, w), table.dtype),
            pltpu.SemaphoreType.DMA,
            pltpu.SemaphoreType.DMA,
        ],
    )(table, idx)


SCH = 80
N_PAD = 10240
STRIPE = N_PAD // NS


def _sc_scatter_body(msg_hbm, dst_hbm, zeros_hbm, out_hbm, idx_v, rows_v, acc_sh):
    c = lax.axis_index("c")
    s = lax.axis_index("s")
    per_core = N_EDGES // NC
    per_tile = per_core // NS
    base = c * per_core + s * per_tile
    n_sub = STRIPE // SCH

    pltpu.sync_copy(zeros_hbm, rows_v)

    @pl.loop(0, n_sub)
    def _(k):
        off = pl.multiple_of(s * STRIPE + k * SCH, 8)
        pltpu.sync_copy(rows_v, acc_sh.at[pl.ds(off, SCH)])

    plsc.subcore_barrier()

    @pl.loop(0, per_tile // SCH)
    def _(j):
        off = pl.multiple_of(base + j * SCH, 8)
        pltpu.sync_copy(dst_hbm.at[pl.ds(off, SCH)], idx_v)
        pltpu.sync_copy(msg_hbm.at[pl.ds(off, SCH)], rows_v)
        pltpu.sync_copy(rows_v, acc_sh.at[idx_v], add=True)

    plsc.subcore_barrier()

    @pl.loop(0, n_sub)
    def _(k):
        off = pl.multiple_of(s * STRIPE + k * SCH, 8)
        pltpu.sync_copy(acc_sh.at[pl.ds(off, SCH)], rows_v)
        pltpu.sync_copy(rows_v, out_hbm.at[c, pl.ds(off, SCH)])


def _sc_segment_sum(msg, dst_idx, zeros_stripe):
    mesh = plsc.VectorSubcoreMesh(core_axis_name="c", subcore_axis_name="s")
    return pl.kernel(
        _sc_scatter_body,
        out_type=jax.ShapeDtypeStruct((NC, N_PAD, D), jnp.float32),
        mesh=mesh,
        scratch_types=[
            pltpu.VMEM((SCH,), jnp.int32),
            pltpu.VMEM((SCH, D), jnp.float32),
            pltpu.VMEM_SHARED((N_PAD, D), jnp.float32),
        ],
    )(msg, dst_idx, zeros_stripe)


BLK = 512


def _silu(x):
    return x * jax.nn.sigmoid(x)


def _unpack_bf16_pair(x_i32):
    f32 = jnp.float32
    bf = jnp.bfloat16
    even = lax.bitcast_convert_type(jnp.left_shift(x_i32, 16), f32)
    odd = lax.bitcast_convert_type(
        jnp.bitwise_and(x_i32, jnp.int32(-65536)), f32)
    return even.astype(bf), odd.astype(bf)


def _sigmoid_t(x):
    return 0.5 * jnp.tanh(x * 0.5) + 0.5


def _silu_t(x):
    return x * _sigmoid_t(x)


def _mlp_body(src, dst, ef, ew,
              w0sd, w0e, b0, gw1, gb1, ow1, ob1, gw2, gb2, ow2, ob2,
              msg_out):
    f32 = jnp.float32
    bf = jnp.bfloat16
    es, osrc = _unpack_bf16_pair(src[...])
    ed, od = _unpack_bf16_pair(dst[...])
    x = jnp.concatenate([es, osrc, ed, od], axis=1)
    pre0 = (jnp.dot(x, w0sd[...], preferred_element_type=f32)
            + jnp.dot(ef[...], w0e[...], preferred_element_type=f32)
            + b0[...])
    a1 = _silu_t(pre0.astype(bf))
    g1 = _silu_t((jnp.dot(a1[:, :H], gw1[...], preferred_element_type=f32) + gb1[...]).astype(bf))
    o1 = _silu_t((jnp.dot(a1[:, H:], ow1[...], preferred_element_type=f32) + ob1[...]).astype(bf))
    gp = (jnp.dot(g1, gw2[...], preferred_element_type=f32) + gb2[...]).astype(bf)
    op = (jnp.dot(o1, ow2[...], preferred_element_type=f32) + ob2[...]).astype(bf)
    msg_out[...] = (_silu_t(op) * _sigmoid_t(gp)).astype(f32) * ew[...]


def _edge_mlp(gathered, edge_feat, edge_weight, weights):
    n_blocks = N_EDGES // BLK
    dst_block_off = N_EDGES // BLK

    def full(w):
        return pl.BlockSpec(w.shape, lambda i: tuple(0 for _ in w.shape))

    w_specs = [full(w) for w in weights]
    return pl.pallas_call(
        _mlp_body,
        grid=(n_blocks,),
        in_specs=[
            pl.BlockSpec((BLK, D // 2), lambda i: (i, 0)),
            pl.BlockSpec((BLK, D // 2), lambda i: (i + dst_block_off, 0)),
            pl.BlockSpec((BLK, ED), lambda i: (i, 0)),
            pl.BlockSpec((BLK, D), lambda i: (i, 0)),
            *w_specs,
        ],
        out_specs=pl.BlockSpec((BLK, D), lambda i: (i, 0)),
        out_shape=jax.ShapeDtypeStruct((N_EDGES, D), jnp.float32),
    )(gathered, gathered, edge_feat, edge_weight, *weights)


NBLK = 2000


def _final_body(node_feat, partials, lin_w, lin_b, out):
    agg = partials[0] + partials[1]
    out[...] = node_feat[...] + jnp.dot(
        agg, lin_w[...], preferred_element_type=jnp.float32) + lin_b[...]


def _final_linear(node_feat, partials, lin_w, lin_b):
    n_blocks = N_NODES // NBLK
    return pl.pallas_call(
        _final_body,
        grid=(n_blocks,),
        in_specs=[
            pl.BlockSpec((NBLK, D), lambda i: (i, 0)),
            pl.BlockSpec((NC, NBLK, D), lambda i: (0, i, 0)),
            pl.BlockSpec((D, D), lambda i: (0, 0)),
            pl.BlockSpec((1, D), lambda i: (0, 0)),
        ],
        out_specs=pl.BlockSpec((NBLK, D), lambda i: (i, 0)),
        out_shape=jax.ShapeDtypeStruct((N_NODES, D), jnp.float32),
    )(node_feat, partials, lin_w, lin_b)


def kernel(node_feat, edge_feat, edge_weight, edge_index,
           g_W0, g_b0, g_W1, g_b1, g_W2, g_b2,
           o_W0, o_b0, o_W1, o_b1, o_W2, o_b2,
           lin_W, lin_b):
    idx_flat = edge_index.reshape(-1).astype(jnp.int32)
    b_pad = NW * GCH * ((2 * N_EDGES + NW * GCH - 1) // (NW * GCH))
    idx_pad = jnp.concatenate(
        [idx_flat, jnp.zeros((b_pad - 2 * N_EDGES,), jnp.int32)])
    bf = jnp.bfloat16
    node_packed = lax.bitcast_convert_type(
        node_feat.astype(bf).reshape(N_NODES, D // 2, 2), jnp.int32)
    gathered = _sc_gather(node_packed, idx_pad)

    w0 = jnp.concatenate([g_W0, o_W0], axis=1)
    w0s, w0d, w0e = w0[:D], w0[D:2 * D], w0[2 * D:]
    w0sd = jnp.concatenate(
        [w0s[0::2], w0s[1::2], w0d[0::2], w0d[1::2]], axis=0)
    b0 = jnp.concatenate([g_b0, o_b0]).reshape(1, 2 * H)
    weights = [w0sd.astype(bf), w0e.astype(bf), b0,
               g_W1.astype(bf), g_b1.reshape(1, H), o_W1.astype(bf), o_b1.reshape(1, H),
               g_W2.astype(bf), g_b2.reshape(1, D), o_W2.astype(bf), o_b2.reshape(1, D)]

    msg = _edge_mlp(gathered, edge_feat.astype(bf), edge_weight, weights)

    dst_idx = edge_index[1].astype(jnp.int32)
    zeros_stripe = jnp.zeros((SCH, D), jnp.float32)
    partials = _sc_segment_sum(msg, dst_idx, zeros_stripe)

    return _final_linear(node_feat, partials, lin_W, lin_b.reshape(1, D))


# speedup vs baseline: 2.6009x; 1.0558x over previous
import functools

import jax
import jax.numpy as jnp
from jax import lax
from jax.experimental import pallas as pl
from jax.experimental.pallas import tpu as pltpu
from jax.experimental.pallas import tpu_sc as plsc

N_NODES = 10000
N_EDGES = 320000
D = 128
ED = 16
H = 256

NC = 2
NS = 16
NW = NC * NS

GCH = 128


def _sc_gather_body(table_hbm, idx_hbm, out_hbm,
                    idx_a, idx_b, rows_a, rows_b, sem_a, sem_b):
    c = lax.axis_index("c")
    s = lax.axis_index("s")
    wid = s * NC + c
    n_total = idx_hbm.shape[0]
    per_w = n_total // NW
    base = wid * per_w
    n_pairs = per_w // GCH // 2

    def chunk(j):
        return pl.ds(pl.multiple_of(base + j * GCH, GCH), GCH)

    pltpu.sync_copy(idx_hbm.at[chunk(0)], idx_a)
    pltpu.async_copy(table_hbm.at[idx_a], rows_a, sem_a)

    @pl.loop(0, n_pairs)
    def _(jj):
        j = jj * 2
        pltpu.sync_copy(idx_hbm.at[chunk(j + 1)], idx_b)
        pltpu.async_copy(table_hbm.at[idx_b], rows_b, sem_b)
        pltpu.make_async_copy(table_hbm.at[idx_a], rows_a, sem_a).wait()
        pltpu.sync_copy(rows_a, out_hbm.at[chunk(j)])

        @pl.when(jj < n_pairs - 1)
        def _():
            pltpu.sync_copy(idx_hbm.at[chunk(j + 2)], idx_a)
            pltpu.async_copy(table_hbm.at[idx_a], rows_a, sem_a)

        pltpu.make_async_copy(table_hbm.at[idx_b], rows_b, sem_b).wait()
        pltpu.sync_copy(rows_b, out_hbm.at[chunk(j + 1)])


TW = 64


def _sc_gather(table, idx):
    b = idx.shape[0]
    mesh = plsc.VectorSubcoreMesh(core_axis_name="c", subcore_axis_name="s")
    return pl.kernel(
        _sc_gather_body,
        out_type=jax.ShapeDtypeStruct((b, TW), jnp.int32),
        mesh=mesh,
        compiler_params=pltpu.CompilerParams(use_tc_tiling_on_sc=False),
        scratch_types=[
            pltpu.VMEM((GCH,), jnp.int32),
            pltpu.VMEM((GCH,), jnp.int32),
            pltpu.VMEM((GCH, TW), jnp.int32),
            pltpu.VMEM((GCH, TW), jnp.int32),
            pltpu.SemaphoreType.DMA,
            pltpu.SemaphoreType.DMA,
        ],
    )(table, idx)


SCH = 80
N_PAD = 10240
STRIPE = N_PAD // NS


def _sc_scatter_body(msg_hbm, dst_hbm, zeros_hbm, out_hbm, idx_v, rows_v, acc_sh):
    c = lax.axis_index("c")
    s = lax.axis_index("s")
    per_core = N_EDGES // NC
    per_tile = per_core // NS
    base = c * per_core + s * per_tile
    n_sub = STRIPE // SCH

    pltpu.sync_copy(zeros_hbm, rows_v)

    @pl.loop(0, n_sub)
    def _(k):
        off = pl.multiple_of(s * STRIPE + k * SCH, 8)
        pltpu.sync_copy(rows_v, acc_sh.at[pl.ds(off, SCH)])

    plsc.subcore_barrier()

    @pl.loop(0, per_tile // SCH)
    def _(j):
        off = pl.multiple_of(base + j * SCH, 8)
        pltpu.sync_copy(dst_hbm.at[pl.ds(off, SCH)], idx_v)
        pltpu.sync_copy(msg_hbm.at[pl.ds(off, SCH)], rows_v)
        pltpu.sync_copy(rows_v, acc_sh.at[idx_v], add=True)

    plsc.subcore_barrier()

    @pl.loop(0, n_sub)
    def _(k):
        off = pl.multiple_of(s * STRIPE + k * SCH, 8)
        pltpu.sync_copy(acc_sh.at[pl.ds(off, SCH)], rows_v)
        pltpu.sync_copy(rows_v, out_hbm.at[c, pl.ds(off, SCH)])


def _sc_segment_sum(msg, dst_idx, zeros_stripe):
    mesh = plsc.VectorSubcoreMesh(core_axis_name="c", subcore_axis_name="s")
    return pl.kernel(
        _sc_scatter_body,
        out_type=jax.ShapeDtypeStruct((NC, N_PAD, D), jnp.float32),
        mesh=mesh,
        scratch_types=[
            pltpu.VMEM((SCH,), jnp.int32),
            pltpu.VMEM((SCH, D), jnp.float32),
            pltpu.VMEM_SHARED((N_PAD, D), jnp.float32),
        ],
    )(msg, dst_idx, zeros_stripe)


BLK = 1280


def _silu(x):
    return x * jax.nn.sigmoid(x)


def _unpack_pairs(x_i32):
    f32 = jnp.float32
    bf = jnp.bfloat16
    even = lax.bitcast_convert_type(jnp.left_shift(x_i32, 16), f32).astype(bf)
    odd = lax.bitcast_convert_type(
        jnp.bitwise_and(x_i32, jnp.int32(-65536)), f32).astype(bf)
    return jnp.concatenate([even, odd], axis=1)


def _sigmoid_t(x):
    return 0.5 * jnp.tanh(x * 0.5) + 0.5


def _silu_t(x):
    return x * _sigmoid_t(x)


def _mlp_body(src, dst, ef, ew,
              w0sd, w0e, b0, gw1, gb1, ow1, ob1, gw2, gb2, ow2, ob2,
              msg_out):
    f32 = jnp.float32
    bf = jnp.bfloat16
    x = jnp.concatenate(
        [_unpack_pairs(src[...]), _unpack_pairs(dst[...])], axis=1)
    pre0 = (jnp.dot(x, w0sd[...], preferred_element_type=f32)
            + jnp.dot(ef[...], w0e[...], preferred_element_type=f32)
            + b0[...])
    a1 = _silu_t(pre0.astype(bf))
    g1 = _silu_t((jnp.dot(a1[:, :H], gw1[...], preferred_element_type=f32) + gb1[...]).astype(bf))
    o1 = _silu_t((jnp.dot(a1[:, H:], ow1[...], preferred_element_type=f32) + ob1[...]).astype(bf))
    gp = (jnp.dot(g1, gw2[...], preferred_element_type=f32) + gb2[...]).astype(bf)
    op = (jnp.dot(o1, ow2[...], preferred_element_type=f32) + ob2[...]).astype(bf)
    msg_out[...] = (_silu_t(op) * _sigmoid_t(gp)).astype(f32) * ew[...]


def _edge_mlp(gathered, edge_feat, edge_weight, weights):
    n_blocks = N_EDGES // BLK
    dst_block_off = N_EDGES // BLK

    def full(w):
        return pl.BlockSpec(w.shape, lambda i: tuple(0 for _ in w.shape))

    w_specs = [full(w) for w in weights]
    return pl.pallas_call(
        _mlp_body,
        grid=(n_blocks,),
        in_specs=[
            pl.BlockSpec((BLK, TW), lambda i: (i, 0)),
            pl.BlockSpec((BLK, TW), lambda i: (i + dst_block_off, 0)),
            pl.BlockSpec((BLK, ED), lambda i: (i, 0)),
            pl.BlockSpec((BLK, D), lambda i: (i, 0)),
            *w_specs,
        ],
        out_specs=pl.BlockSpec((BLK, D), lambda i: (i, 0)),
        out_shape=jax.ShapeDtypeStruct((N_EDGES, D), jnp.float32),
    )(gathered, gathered, edge_feat, edge_weight, *weights)


NBLK = 2000


def _final_body(node_feat, partials, lin_w, lin_b, out):
    agg = partials[0] + partials[1]
    out[...] = node_feat[...] + jnp.dot(
        agg, lin_w[...], preferred_element_type=jnp.float32) + lin_b[...]


def _final_linear(node_feat, partials, lin_w, lin_b):
    n_blocks = N_NODES // NBLK
    return pl.pallas_call(
        _final_body,
        grid=(n_blocks,),
        in_specs=[
            pl.BlockSpec((NBLK, D), lambda i: (i, 0)),
            pl.BlockSpec((NC, NBLK, D), lambda i: (0, i, 0)),
            pl.BlockSpec((D, D), lambda i: (0, 0)),
            pl.BlockSpec((1, D), lambda i: (0, 0)),
        ],
        out_specs=pl.BlockSpec((NBLK, D), lambda i: (i, 0)),
        out_shape=jax.ShapeDtypeStruct((N_NODES, D), jnp.float32),
    )(node_feat, partials, lin_w, lin_b)


def kernel(node_feat, edge_feat, edge_weight, edge_index,
           g_W0, g_b0, g_W1, g_b1, g_W2, g_b2,
           o_W0, o_b0, o_W1, o_b1, o_W2, o_b2,
           lin_W, lin_b):
    idx_flat = edge_index.reshape(-1).astype(jnp.int32)
    grp = NW * GCH * 2
    b_pad = grp * ((2 * N_EDGES + grp - 1) // grp)
    idx_pad = jnp.concatenate(
        [idx_flat, jnp.zeros((b_pad - 2 * N_EDGES,), jnp.int32)])
    bf = jnp.bfloat16
    node_packed = lax.bitcast_convert_type(
        node_feat.astype(bf).reshape(N_NODES, D // 2, 2), jnp.int32)
    gathered = _sc_gather(node_packed, idx_pad)

    w0 = jnp.concatenate([g_W0, o_W0], axis=1)
    w0s, w0d, w0e = w0[:D], w0[D:2 * D], w0[2 * D:]
    w0sd = jnp.concatenate(
        [w0s[0::2], w0s[1::2], w0d[0::2], w0d[1::2]], axis=0)
    b0 = jnp.concatenate([g_b0, o_b0]).reshape(1, 2 * H)
    weights = [w0sd.astype(bf), w0e.astype(bf), b0,
               g_W1.astype(bf), g_b1.reshape(1, H), o_W1.astype(bf), o_b1.reshape(1, H),
               g_W2.astype(bf), g_b2.reshape(1, D), o_W2.astype(bf), o_b2.reshape(1, D)]

    msg = _edge_mlp(gathered, edge_feat.astype(bf), edge_weight, weights)

    dst_idx = edge_index[1].astype(jnp.int32)
    zeros_stripe = jnp.zeros((SCH, D), jnp.float32)
    partials = _sc_segment_sum(msg, dst_idx, zeros_stripe)

    return _final_linear(node_feat, partials, lin_W, lin_b.reshape(1, D))
